# baseline (speedup 1.0000x reference)
import jax
import jax.numpy as jnp
from jax import lax
from jax.experimental import pallas as pl
from jax.experimental.pallas import tpu as pltpu

N = 1000
C = 81
HW = 28 * 28
SHEETS_PER_STEP = 56
STEPS = HW // SHEETS_PER_STEP


def _tc_body(cid_ref, pred_ref, targ_ref, out_ref):
    step = pl.program_id(0)
    cid = cid_ref[...]
    x = pred_ref[...]
    z = targ_ref[...]

    c_iota = lax.broadcasted_iota(jnp.int32, (1, C, N), 1)
    cid_pos = jnp.where(cid > 0, cid, -1)
    onehot = (cid_pos[:, None, :] == c_iota).astype(jnp.float32)
    y = jnp.sum(x * onehot, axis=1)

    bce = jnp.maximum(y, 0.0) - y * z + jnp.log1p(jnp.exp(-jnp.abs(y)))
    step_sum = jnp.sum(bce).reshape(1, 1)

    @pl.when(step == 0)
    def _():
        out_ref[...] = jnp.zeros_like(out_ref)

    out_ref[0:1, 0:1] += step_sum

    @pl.when(step == STEPS - 1)
    def _():
        npos = jnp.sum((cid > 0).astype(jnp.float32))
        ln2 = jnp.log1p(jnp.exp(-jnp.abs(jnp.float32(0.0))))
        total = out_ref[0, 0] - (float(N) - npos) * float(HW) * ln2
        denom = jnp.maximum(npos, 1.0) * float(HW)
        out_ref[...] = (total / denom).reshape(1, 1)


def kernel(target_masks, target_class_ids, pred_masks):
    cid = target_class_ids.astype(jnp.int32).reshape(1, N)
    predt = jnp.transpose(pred_masks, (2, 3, 1, 0)).reshape(HW, C, N)
    targt = jnp.transpose(target_masks, (1, 2, 0)).reshape(HW, N)
    loss = pl.pallas_call(
        _tc_body,
        grid=(STEPS,),
        in_specs=[
            pl.BlockSpec((1, N), lambda s: (0, 0)),
            pl.BlockSpec((SHEETS_PER_STEP, C, N), lambda s: (s, 0, 0)),
            pl.BlockSpec((SHEETS_PER_STEP, N), lambda s: (s, 0)),
        ],
        out_specs=pl.BlockSpec((1, 1), lambda s: (0, 0)),
        out_shape=jax.ShapeDtypeStruct((1, 1), jnp.float32),
        compiler_params=pltpu.CompilerParams(
            dimension_semantics=("arbitrary",),
            vmem_limit_bytes=100 * 1024 * 1024),
    )(cid, predt, targt)
    return loss[0, 0]


# speedup vs baseline: 1.0489x; 1.0489x over previous
import jax
import jax.numpy as jnp
from jax import lax
from jax.experimental import pallas as pl
from jax.experimental.pallas import tpu as pltpu

N = 1000
C = 81
HW = 28 * 28
SHEETS_PER_STEP = 56
STEPS = HW // SHEETS_PER_STEP


def _tc_body(cid_ref, pred_ref, targ_ref, out_ref):
    step = pl.program_id(0)
    cid = cid_ref[...]
    x = pred_ref[...]
    z = targ_ref[...].reshape(SHEETS_PER_STEP, N)

    c_iota = lax.broadcasted_iota(jnp.int32, (1, C, N), 1)
    cid_pos = jnp.where(cid > 0, cid, -1)
    onehot = (cid_pos[:, None, :] == c_iota).astype(jnp.float32)
    y = jnp.sum(x * onehot, axis=1)

    bce = jnp.maximum(y, 0.0) - y * z + jnp.log1p(jnp.exp(-jnp.abs(y)))
    step_sum = jnp.sum(bce).reshape(1, 1)

    @pl.when(step == 0)
    def _():
        out_ref[...] = jnp.zeros_like(out_ref)

    out_ref[0:1, 0:1] += step_sum

    @pl.when(step == STEPS - 1)
    def _():
        npos = jnp.sum((cid > 0).astype(jnp.float32))
        ln2 = jnp.log1p(jnp.exp(-jnp.abs(jnp.float32(0.0))))
        total = out_ref[0, 0] - (float(N) - npos) * float(HW) * ln2
        denom = jnp.maximum(npos, 1.0) * float(HW)
        out_ref[...] = (total / denom).reshape(1, 1)


def kernel(target_masks, target_class_ids, pred_masks):
    cid = target_class_ids.astype(jnp.int32).reshape(1, N)
    predt = jnp.transpose(pred_masks, (2, 3, 1, 0)).reshape(HW, C, N)
    targt = jnp.transpose(target_masks, (1, 2, 0))
    loss = pl.pallas_call(
        _tc_body,
        grid=(STEPS,),
        in_specs=[
            pl.BlockSpec((1, N), lambda s: (0, 0)),
            pl.BlockSpec((SHEETS_PER_STEP, C, N), lambda s: (s, 0, 0)),
            pl.BlockSpec((SHEETS_PER_STEP // 28, 28, N), lambda s: (s, 0, 0)),
        ],
        out_specs=pl.BlockSpec((1, 1), lambda s: (0, 0)),
        out_shape=jax.ShapeDtypeStruct((1, 1), jnp.float32),
        compiler_params=pltpu.CompilerParams(
            dimension_semantics=("arbitrary",),
            vmem_limit_bytes=100 * 1024 * 1024),
    )(cid, predt, targt)
    return loss[0, 0]
